# Initial kernel scaffold; baseline (speedup 1.0000x reference)
import jax
import jax.numpy as jnp
from jax.experimental import pallas as pl


def kernel(x, adj, nodes, epoch, W0, b0, W1, b1):
    raise NotImplementedError("write your pallas kernel here")


# speedup vs baseline: 10.1099x; 10.1099x over previous
import functools

import jax
import jax.numpy as jnp
from jax import lax
from jax.experimental import pallas as pl
from jax.experimental.pallas import tpu as pltpu
from jax.experimental.pallas import tpu_sc as plsc

N = 10000
E = 320000
NFEAT = 128
NHID = 128
NCLASS = 64
DH = 64

NC = 2
NS = 16
NW = NC * NS
CH = 80
GRP = 5
RPT = 624
REM = N - NS * RPT

_MESH = plsc.VectorSubcoreMesh(core_axis_name="c", subcore_axis_name="s")


def _make_scatter(feature_split):
  ept = E // NS if feature_split else E // NW
  nch = ept // CH
  ngrp = nch // GRP

  @functools.partial(
      pl.kernel,
      out_type=jax.ShapeDtypeStruct((NC, N, DH), jnp.float32),
      mesh=_MESH,
      compiler_params=pltpu.CompilerParams(use_tc_tiling_on_sc=False),
      scratch_types=[
          pltpu.VMEM((nch, CH), jnp.int32),
          pltpu.VMEM((nch, CH), jnp.int32),
          [pltpu.VMEM((CH, DH), jnp.float32) for _ in range(GRP)],
          pltpu.VMEM_SHARED((N, DH), jnp.float32),
          pltpu.SemaphoreType.DMA,
      ],
  )
  def scatter_kernel(h_hbm, src_hbm, dst_hbm, zeros_hbm, out_hbm,
                     src_v, dst_v, rows, acc, sem):
    c = lax.axis_index("c")
    s = lax.axis_index("s")
    eslice = s if feature_split else c * NS + s
    gref = h_hbm.at[c] if feature_split else h_hbm
    pltpu.sync_copy(src_hbm.at[eslice], src_v)
    pltpu.sync_copy(dst_hbm.at[eslice], dst_v)
    r0 = s * RPT
    pltpu.sync_copy(zeros_hbm.at[pl.ds(r0, RPT)], acc.at[pl.ds(r0, RPT)])

    @pl.when(s == 0)
    def _():
      pltpu.sync_copy(zeros_hbm.at[pl.ds(NS * RPT, REM)],
                      acc.at[pl.ds(NS * RPT, REM)])

    plsc.subcore_barrier()

    def group(g, _):
      base = g * GRP
      descs = []
      for j in range(GRP):
        descs.append(
            pltpu.async_copy(gref.at[src_v.at[base + j]], rows[j], sem))
      for j in range(GRP):
        descs[j].wait()
        pltpu.sync_copy(rows[j], acc.at[dst_v.at[base + j]], add=True)
      return 0

    lax.fori_loop(0, ngrp, group, 0)
    plsc.subcore_barrier()
    pltpu.sync_copy(acc.at[pl.ds(r0, RPT)], out_hbm.at[c, pl.ds(r0, RPT)])

    @pl.when(s == 0)
    def _():
      pltpu.sync_copy(acc.at[pl.ds(NS * RPT, REM)],
                      out_hbm.at[c, pl.ds(NS * RPT, REM)])

  return scatter_kernel


_scatter1 = _make_scatter(True)
_scatter2 = _make_scatter(False)

_BM = 1000


def _mm1_body(x_ref, w_ref, b_ref, o_ref):
  r = (jnp.dot(x_ref[...], w_ref[...], preferred_element_type=jnp.float32)
       + b_ref[...])
  o_ref[0] = r[:, :DH]
  o_ref[1] = r[:, DH:]


def _mid_body(p_ref, w_ref, b_ref, o_ref):
  h = jnp.concatenate([p_ref[0], p_ref[1]], axis=1)
  nrm = jnp.sqrt(jnp.sum(h * h, axis=1, keepdims=True))
  z = h / jnp.maximum(nrm, 1e-12)
  h1 = jnp.maximum(z, 0.0)
  o_ref[...] = (
      jnp.dot(h1, w_ref[...], preferred_element_type=jnp.float32)
      + b_ref[...])


def _lsm_body(q_ref, o_ref):
  h = q_ref[0] + q_ref[1]
  m = jnp.max(h, axis=1, keepdims=True)
  e = jnp.exp(h - m)
  lse = jnp.log(jnp.sum(e, axis=1, keepdims=True))
  o_ref[...] = h - m - lse


def _mm1(x, W0, b0):
  return pl.pallas_call(
      _mm1_body,
      grid=(N // _BM,),
      in_specs=[
          pl.BlockSpec((_BM, NFEAT), lambda i: (i, 0)),
          pl.BlockSpec((NFEAT, NHID), lambda i: (0, 0)),
          pl.BlockSpec((1, NHID), lambda i: (0, 0)),
      ],
      out_specs=pl.BlockSpec((NC, _BM, DH), lambda i: (0, i, 0)),
      out_shape=jax.ShapeDtypeStruct((NC, N, DH), jnp.float32),
  )(x, W0, b0)


def _mid(p, W1, b1):
  return pl.pallas_call(
      _mid_body,
      grid=(N // _BM,),
      in_specs=[
          pl.BlockSpec((NC, _BM, DH), lambda i: (0, i, 0)),
          pl.BlockSpec((NHID, NCLASS), lambda i: (0, 0)),
          pl.BlockSpec((1, NCLASS), lambda i: (0, 0)),
      ],
      out_specs=pl.BlockSpec((_BM, NCLASS), lambda i: (i, 0)),
      out_shape=jax.ShapeDtypeStruct((N, NCLASS), jnp.float32),
  )(p, W1, b1)


def _lsm(q):
  return pl.pallas_call(
      _lsm_body,
      grid=(N // _BM,),
      in_specs=[pl.BlockSpec((NC, _BM, NCLASS), lambda i: (0, i, 0))],
      out_specs=pl.BlockSpec((_BM, NCLASS), lambda i: (i, 0)),
      out_shape=jax.ShapeDtypeStruct((N, NCLASS), jnp.float32),
  )(q)


def kernel(x, adj, nodes, epoch, W0, b0, W1, b1):
  src16 = adj[0].reshape(NS, (E // NS) // CH, CH)
  dst16 = adj[1].reshape(NS, (E // NS) // CH, CH)
  src32 = adj[0].reshape(NW, (E // NW) // CH, CH)
  dst32 = adj[1].reshape(NW, (E // NW) // CH, CH)
  zeros = jnp.zeros((N, DH), jnp.float32)
  h = _mm1(x, W0, b0)
  p1 = _scatter1(h, src16, dst16, zeros)
  h2 = _mid(p1, W1, b1)
  p2 = _scatter2(h2, src32, dst32, zeros)
  return _lsm(p2)


# speedup vs baseline: 12.2000x; 1.2067x over previous
import functools

import jax
import jax.numpy as jnp
from jax import lax
from jax.experimental import pallas as pl
from jax.experimental.pallas import tpu as pltpu
from jax.experimental.pallas import tpu_sc as plsc

N = 10000
E = 320000
NFEAT = 128
NHID = 128
NCLASS = 64
DH = 64

NC = 2
NS = 16
NW = NC * NS
CH = 80
GRP = 5
NBANK = 2
IBANK = 4
GCH = GRP * CH
RPT = 624
REM = N - NS * RPT

_MESH = plsc.VectorSubcoreMesh(core_axis_name="c", subcore_axis_name="s")


def _make_scatter(feature_split):
  ept = E // NS if feature_split else E // NW
  nch = ept // CH
  ngrp = nch // GRP

  @functools.partial(
      pl.kernel,
      out_type=jax.ShapeDtypeStruct((NC, N, DH), jnp.float32),
      mesh=_MESH,
      compiler_params=pltpu.CompilerParams(use_tc_tiling_on_sc=False),
      scratch_types=[
          [pltpu.VMEM((GCH,), jnp.int32) for _ in range(IBANK)],
          [pltpu.VMEM((GCH,), jnp.int32) for _ in range(IBANK)],
          [pltpu.VMEM((CH, DH), jnp.float32)
           for _ in range(NBANK * GRP)],
          pltpu.VMEM_SHARED((N, DH), jnp.float32),
          pltpu.SemaphoreType.DMA,
          pltpu.SemaphoreType.DMA,
          pltpu.SemaphoreType.DMA,
      ],
  )
  def scatter_kernel(h_hbm, adj_hbm, zeros_hbm, out_hbm,
                     src_v, dst_v, rows, acc, gsem, ssem, isem):
    c = lax.axis_index("c")
    s = lax.axis_index("s")
    e0 = (s if feature_split else c * NS + s) * ept
    gref = h_hbm.at[c] if feature_split else h_hbm

    def stage_idx(g, bank):
      pltpu.async_copy(
          adj_hbm.at[0, pl.ds(e0 + g * GCH, GCH)], src_v[bank], isem)
      pltpu.async_copy(
          adj_hbm.at[1, pl.ds(e0 + g * GCH, GCH)], dst_v[bank], isem)

    def wait_idx():
      for _ in range(2):
        pltpu.make_async_copy(
            adj_hbm.at[0, pl.ds(e0, GCH)], src_v[0], isem).wait()

    r0 = s * RPT
    pltpu.sync_copy(zeros_hbm.at[pl.ds(r0, RPT)], acc.at[pl.ds(r0, RPT)])

    @pl.when(s == 0)
    def _():
      pltpu.sync_copy(zeros_hbm.at[pl.ds(NS * RPT, REM)],
                      acc.at[pl.ds(NS * RPT, REM)])

    stage_idx(0, 0)
    stage_idx(1, 1)
    plsc.subcore_barrier()

    def drain(n):
      for _ in range(n):
        pltpu.make_async_copy(
            zeros_hbm.at[pl.ds(0, CH)], rows[0], ssem).wait()

    def run_group(rbank, ibank):
      gathers = []
      for j in range(GRP):
        gathers.append(pltpu.async_copy(
            gref.at[src_v[ibank].at[pl.ds(j * CH, CH)]],
            rows[rbank * GRP + j], gsem))
      for j in range(GRP):
        gathers[j].wait()
        pltpu.async_copy(
            rows[rbank * GRP + j],
            acc.at[dst_v[ibank].at[pl.ds(j * CH, CH)]],
            ssem, add=True)

    def quad(i, _):
      for t in range(4):
        wait_idx()
        if t < 2:
          @pl.when(i >= 1)
          def _():
            drain(GRP)
        else:
          drain(GRP)
        st = 4 * i + t + 2

        @pl.when(st < ngrp)
        def _():
          stage_idx(st, (t + 2) % 4)

        run_group(t % 2, t)
      return 0

    nquad = ngrp // 4
    lax.fori_loop(0, nquad, quad, 0)
    for t in range(ngrp % 4):
      g = 4 * nquad + t
      wait_idx()
      drain(GRP)
      if g + 2 < ngrp:
        stage_idx(g + 2, (t + 2) % 4)
      run_group(t % 2, t)
    drain(NBANK * GRP)
    plsc.subcore_barrier()
    pltpu.sync_copy(acc.at[pl.ds(r0, RPT)], out_hbm.at[c, pl.ds(r0, RPT)])

    @pl.when(s == 0)
    def _():
      pltpu.sync_copy(acc.at[pl.ds(NS * RPT, REM)],
                      out_hbm.at[c, pl.ds(NS * RPT, REM)])

  return scatter_kernel


_scatter1 = _make_scatter(True)
_scatter2 = _make_scatter(False)


def _mm1_body(x_ref, w_ref, b_ref, o_ref):
  r = (jnp.dot(x_ref[...], w_ref[...], preferred_element_type=jnp.float32)
       + b_ref[...])
  o_ref[0] = r[:, :DH]
  o_ref[1] = r[:, DH:]


def _mid_body(p_ref, w_ref, b_ref, o_ref):
  h = jnp.concatenate([p_ref[0], p_ref[1]], axis=1)
  nrm = jnp.sqrt(jnp.sum(h * h, axis=1, keepdims=True))
  z = h / jnp.maximum(nrm, 1e-12)
  h1 = jnp.maximum(z, 0.0)
  o_ref[...] = (
      jnp.dot(h1, w_ref[...], preferred_element_type=jnp.float32)
      + b_ref[...])


def _lsm_body(q_ref, o_ref):
  h = q_ref[0] + q_ref[1]
  m = jnp.max(h, axis=1, keepdims=True)
  e = jnp.exp(h - m)
  lse = jnp.log(jnp.sum(e, axis=1, keepdims=True))
  o_ref[...] = h - m - lse


def _mm1(x, W0, b0):
  return pl.pallas_call(
      _mm1_body,
      out_shape=jax.ShapeDtypeStruct((NC, N, DH), jnp.float32),
  )(x, W0, b0)


def _mid(p, W1, b1):
  return pl.pallas_call(
      _mid_body,
      out_shape=jax.ShapeDtypeStruct((N, NCLASS), jnp.float32),
  )(p, W1, b1)


def _lsm(q):
  return pl.pallas_call(
      _lsm_body,
      out_shape=jax.ShapeDtypeStruct((N, NCLASS), jnp.float32),
  )(q)


def kernel(x, adj, nodes, epoch, W0, b0, W1, b1):
  zeros = jnp.zeros((N, DH), jnp.float32)
  h = _mm1(x, W0, b0)
  p1 = _scatter1(h, adj, zeros)
  h2 = _mid(p1, W1, b1)
  p2 = _scatter2(h2, adj, zeros)
  return _lsm(p2)


# speedup vs baseline: 12.4002x; 1.0164x over previous
import functools

import jax
import jax.numpy as jnp
from jax import lax
from jax.experimental import pallas as pl
from jax.experimental.pallas import tpu as pltpu
from jax.experimental.pallas import tpu_sc as plsc

N = 10000
E = 320000
NFEAT = 128
NHID = 128
NCLASS = 64
DH = 64

NC = 2
NS = 16
NW = NC * NS
CH = 80
GRP = 5
NBANK = 2
IBANK = 4
GCH = GRP * CH
RPT = 624
REM = N - NS * RPT

_MESH = plsc.VectorSubcoreMesh(core_axis_name="c", subcore_axis_name="s")


def _make_scatter(feature_split):
  ept = E // NS if feature_split else E // NW
  nch = ept // CH
  ngrp = nch // GRP

  @functools.partial(
      pl.kernel,
      out_type=jax.ShapeDtypeStruct((NC, N, DH), jnp.float32),
      mesh=_MESH,
      compiler_params=pltpu.CompilerParams(use_tc_tiling_on_sc=False),
      scratch_types=[
          [pltpu.VMEM((GCH,), jnp.int32) for _ in range(IBANK)],
          [pltpu.VMEM((GCH,), jnp.int32) for _ in range(IBANK)],
          [pltpu.VMEM((CH, DH), jnp.float32)
           for _ in range(NBANK * GRP)],
          pltpu.VMEM_SHARED((N, DH), jnp.float32),
          pltpu.SemaphoreType.DMA,
          pltpu.SemaphoreType.DMA,
          pltpu.SemaphoreType.DMA,
      ],
  )
  def scatter_kernel(h_hbm, adj_hbm, out_hbm,
                     src_v, dst_v, rows, acc, gsem, ssem, isem):
    c = lax.axis_index("c")
    s = lax.axis_index("s")
    e0 = (s if feature_split else c * NS + s) * ept
    gref = h_hbm.at[c] if feature_split else h_hbm
    dummy = (h_hbm.at[0, pl.ds(0, CH)] if feature_split
             else h_hbm.at[pl.ds(0, CH)])

    def stage_idx(g, bank):
      pltpu.async_copy(
          adj_hbm.at[0, pl.ds(e0 + g * GCH, GCH)], src_v[bank], isem)
      pltpu.async_copy(
          adj_hbm.at[1, pl.ds(e0 + g * GCH, GCH)], dst_v[bank], isem)

    def wait_idx():
      for _ in range(2):
        pltpu.make_async_copy(
            adj_hbm.at[0, pl.ds(e0, GCH)], src_v[0], isem).wait()

    stage_idx(0, 0)
    stage_idx(1, 1)

    def zstore(k, _):
      rows[0][k >> 2, pl.ds(lax.rem(k, 4) * 16, 16)] = (
          jnp.zeros((16,), jnp.float32))
      return 0

    lax.fori_loop(0, CH * 4, zstore, 0)
    r0 = s * RPT
    zcopies = []
    for k in range(7):
      zcopies.append(pltpu.async_copy(
          rows[0], acc.at[pl.ds(r0 + k * CH, CH)], gsem))
    zcopies.append(pltpu.async_copy(
        rows[0].at[pl.ds(0, 64)], acc.at[pl.ds(r0 + 7 * CH, 64)], gsem))

    @pl.when(s == 0)
    def _():
      pltpu.async_copy(
          rows[0].at[pl.ds(0, REM)], acc.at[pl.ds(NS * RPT, REM)],
          gsem).wait()

    for zc in zcopies:
      zc.wait()
    plsc.subcore_barrier()

    def drain(n):
      for _ in range(n):
        pltpu.make_async_copy(dummy, rows[0], ssem).wait()

    def run_group(rbank, ibank):
      gathers = []
      for j in range(GRP):
        gathers.append(pltpu.async_copy(
            gref.at[src_v[ibank].at[pl.ds(j * CH, CH)]],
            rows[rbank * GRP + j], gsem))
      for j in range(GRP):
        gathers[j].wait()
        pltpu.async_copy(
            rows[rbank * GRP + j],
            acc.at[dst_v[ibank].at[pl.ds(j * CH, CH)]],
            ssem, add=True)

    def quad(i, _):
      for t in range(4):
        wait_idx()
        if t < 2:
          @pl.when(i >= 1)
          def _():
            drain(GRP)
        else:
          drain(GRP)
        st = 4 * i + t + 2

        @pl.when(st < ngrp)
        def _():
          stage_idx(st, (t + 2) % 4)

        run_group(t % 2, t)
      return 0

    nquad = ngrp // 4
    lax.fori_loop(0, nquad, quad, 0)
    for t in range(ngrp % 4):
      g = 4 * nquad + t
      wait_idx()
      drain(GRP)
      if g + 2 < ngrp:
        stage_idx(g + 2, (t + 2) % 4)
      run_group(t % 2, t)
    drain(NBANK * GRP)
    plsc.subcore_barrier()
    pltpu.sync_copy(acc.at[pl.ds(r0, RPT)], out_hbm.at[c, pl.ds(r0, RPT)])

    @pl.when(s == 0)
    def _():
      pltpu.sync_copy(acc.at[pl.ds(NS * RPT, REM)],
                      out_hbm.at[c, pl.ds(NS * RPT, REM)])

  return scatter_kernel


_scatter1 = _make_scatter(True)
_scatter2 = _make_scatter(False)


def _mm1_body(x_ref, w_ref, b_ref, o_ref):
  r = (jnp.dot(x_ref[...], w_ref[...], preferred_element_type=jnp.float32)
       + b_ref[...])
  o_ref[0] = r[:, :DH]
  o_ref[1] = r[:, DH:]


def _mid_body(p_ref, w_ref, b_ref, o_ref):
  h = jnp.concatenate([p_ref[0], p_ref[1]], axis=1)
  nrm = jnp.sqrt(jnp.sum(h * h, axis=1, keepdims=True))
  z = h / jnp.maximum(nrm, 1e-12)
  h1 = jnp.maximum(z, 0.0)
  o_ref[...] = (
      jnp.dot(h1, w_ref[...], preferred_element_type=jnp.float32)
      + b_ref[...])


def _lsm_body(q_ref, o_ref):
  h = q_ref[0] + q_ref[1]
  m = jnp.max(h, axis=1, keepdims=True)
  e = jnp.exp(h - m)
  lse = jnp.log(jnp.sum(e, axis=1, keepdims=True))
  o_ref[...] = h - m - lse


def _mm1(x, W0, b0):
  return pl.pallas_call(
      _mm1_body,
      out_shape=jax.ShapeDtypeStruct((NC, N, DH), jnp.float32),
  )(x, W0, b0)


def _mid(p, W1, b1):
  return pl.pallas_call(
      _mid_body,
      out_shape=jax.ShapeDtypeStruct((N, NCLASS), jnp.float32),
      compiler_params=pltpu.CompilerParams(
          allow_input_fusion=[True, False, False]),
  )(p, W1, b1)


def _lsm(q):
  return pl.pallas_call(
      _lsm_body,
      out_shape=jax.ShapeDtypeStruct((N, NCLASS), jnp.float32),
      compiler_params=pltpu.CompilerParams(allow_input_fusion=[True]),
  )(q)


def kernel(x, adj, nodes, epoch, W0, b0, W1, b1):
  h = _mm1(x, W0, b0)
  p1 = _scatter1(h, adj)
  h2 = _mid(p1, W1, b1)
  p2 = _scatter2(h2, adj)
  return _lsm(p2)
